# baseline (speedup 1.0000x reference)
import jax
import jax.numpy as jnp
from jax import lax
from jax.experimental import pallas as pl
from jax.experimental.pallas import tpu as pltpu
from jax.experimental.pallas import tpu_sc as plsc

_NC, _NS = 2, 16
_HC = 8
_NCHUNK = 4


def _compute_chunk(zqb, scr, ib):
    w_dim = zqb.shape[1]
    lanes = lax.iota(jnp.int32, 16)
    p1 = (lanes + 1) & 15
    p2 = (lanes + 2) & 15
    p4 = (lanes + 4) & 15

    def perm(x, pv):
        return lax.gather(
            x, pv[:, None],
            lax.GatherDimensionNumbers(
                offset_dims=(), collapsed_slice_dims=(0,),
                start_index_map=(0,),
            ),
            slice_sizes=(1,),
            mode=lax.GatherScatterMode.PROMISE_IN_BOUNDS,
        )

    @plsc.parallel_loop(0, _HC * w_dim, 1, unroll=6)
    def _(v):
        h = lax.shift_right_logical(v, 5)
        w = lax.bitwise_and(v, w_dim - 1)
        for j in range(4):
            sl = pl.ds(j * 16, 16)
            x = zqb[h, w, sl]
            y = 7.0 / (1.0 + jnp.exp(x * -2.0))
            idx = (y + 0.5).astype(jnp.int32)
            zqb[h, w, sl] = idx.astype(jnp.float32) * (2.0 / 7.0) - 1.0
            t = (idx << 3) | perm(idx, p1)
            t = (t << 6) | perm(t, p2)
            t = (t << 12) | perm(t, p4)
            scr[h, w, sl] = t

    @plsc.parallel_loop(0, _HC, 1, unroll=2)
    def _(h):
        hv = jnp.full((16,), h, jnp.int32)
        for cb in range(8):
            dv = jnp.full((16,), (cb >> 1) * 16 + (cb & 1) * 8, jnp.int32)
            for j in range(2):
                wv = lanes + (j * 16)
                ib[cb, h, pl.ds(j * 16, 16)] = plsc.load_gather(
                    scr, [hv, wv, dv]
                )


def _sc_body(zt_hbm, qt_hbm, i_hbm, zqb, scr, ib, zsems, qsems, isems):
    b = lax.axis_index("c") * _NS + lax.axis_index("s")

    def start_in(c, par):
        pltpu.make_async_copy(
            zt_hbm.at[b, pl.ds(c * _HC, _HC)], zqb.at[par], zsems[par]
        ).start()

    def wait_in(par):
        pltpu.make_async_copy(
            zt_hbm.at[b, pl.ds(0, _HC)], zqb.at[par], zsems[par]
        ).wait()

    def start_out(c, par):
        pltpu.make_async_copy(
            zqb.at[par], qt_hbm.at[b, pl.ds(c * _HC, _HC)], qsems[par]
        ).start()
        pltpu.make_async_copy(
            ib.at[par], i_hbm.at[b, pl.ds(0, 8), pl.ds(c * _HC, _HC)],
            isems[par],
        ).start()

    def wait_out_q(par):
        pltpu.make_async_copy(
            zqb.at[par], qt_hbm.at[b, pl.ds(0, _HC)], qsems[par]
        ).wait()

    def wait_out_i(par):
        pltpu.make_async_copy(
            ib.at[par], i_hbm.at[b, pl.ds(0, 8), pl.ds(0, _HC)], isems[par]
        ).wait()

    start_in(0, 0)
    for c in range(_NCHUNK):
        par = c % 2
        if c + 1 < _NCHUNK:
            if c >= 1:
                wait_out_q(1 - par)
            start_in(c + 1, 1 - par)
        wait_in(par)
        if c >= 2:
            wait_out_i(par)
        _compute_chunk(zqb.at[par], scr, ib.at[par])
        start_out(c, par)
    wait_out_q(_NCHUNK % 2)
    wait_out_q((_NCHUNK - 1) % 2)
    wait_out_i(_NCHUNK % 2)
    wait_out_i((_NCHUNK - 1) % 2)


def kernel(z):
    B, D, H, W = z.shape
    zt = jnp.transpose(z, (0, 2, 3, 1))
    mesh = plsc.VectorSubcoreMesh(
        core_axis_name="c", subcore_axis_name="s",
        num_cores=_NC, num_subcores=_NS,
    )
    kfn = pl.kernel(
        _sc_body,
        out_type=(
            jax.ShapeDtypeStruct((B, H, W, D), jnp.float32),
            jax.ShapeDtypeStruct((B, D // 8, H, W), jnp.int32),
        ),
        mesh=mesh,
        scratch_types=[
            pltpu.VMEM((2, _HC, W, D), jnp.float32),
            pltpu.VMEM((_HC, W, D), jnp.int32),
            pltpu.VMEM((2, 8, _HC, W), jnp.int32),
            (pltpu.SemaphoreType.DMA, pltpu.SemaphoreType.DMA),
            (pltpu.SemaphoreType.DMA, pltpu.SemaphoreType.DMA),
            (pltpu.SemaphoreType.DMA, pltpu.SemaphoreType.DMA),
        ],
        compiler_params=pltpu.CompilerParams(
            use_tc_tiling_on_sc=True, needs_layout_passes=False,
        ),
    )
    qt, idx = kfn(zt)
    return jnp.transpose(qt, (0, 3, 1, 2)), idx


# speedup vs baseline: 1.6956x; 1.6956x over previous
import jax
import jax.numpy as jnp
from jax import lax
from jax.experimental import pallas as pl
from jax.experimental.pallas import tpu as pltpu
from jax.experimental.pallas import tpu_sc as plsc

_NC, _NS = 2, 16
_HC = 8
_NCHUNK = 4


def _compute_chunk(zqb, scr, ib):
    w_dim = zqb.shape[1]
    lanes = lax.iota(jnp.int32, 16)
    p1 = (lanes + 1) & 15
    p2 = (lanes + 2) & 15
    p4 = (lanes + 4) & 15

    def perm(x, pv):
        return lax.gather(
            x, pv[:, None],
            lax.GatherDimensionNumbers(
                offset_dims=(), collapsed_slice_dims=(0,),
                start_index_map=(0,),
            ),
            slice_sizes=(1,),
            mode=lax.GatherScatterMode.PROMISE_IN_BOUNDS,
        )

    @plsc.parallel_loop(0, _HC * w_dim, 1, unroll=4)
    def _(v):
        h = lax.shift_right_logical(v, 5)
        w = lax.bitwise_and(v, w_dim - 1)
        for j in range(4):
            sl = pl.ds(j * 16, 16)
            x = zqb[h, w, sl]
            y = 7.0 / (1.0 + jnp.exp(x * -2.0))
            idx = (y + 0.5).astype(jnp.int32)
            zqb[h, w, sl] = idx.astype(jnp.float32) * (2.0 / 7.0) - 1.0
            t = (idx << 3) | perm(idx, p1)
            t = (t << 6) | perm(t, p2)
            t = (t << 12) | perm(t, p4)
            scr[h, w, sl] = t

    @plsc.parallel_loop(0, _HC, 1, unroll=4)
    def _(h):
        hv = jnp.full((16,), h, jnp.int32)
        for cb in range(8):
            dv = jnp.full((16,), (cb >> 1) * 16 + (cb & 1) * 8, jnp.int32)
            for j in range(2):
                wv = lanes + (j * 16)
                ib[cb, h, pl.ds(j * 16, 16)] = plsc.load_gather(
                    scr, [hv, wv, dv]
                )


def _sc_body(zt_hbm, qt_hbm, i_hbm, zqb, scr, ib, zsems, qsems, isems):
    b = lax.axis_index("c") * _NS + lax.axis_index("s")

    def start_in(c, par):
        pltpu.make_async_copy(
            zt_hbm.at[b, pl.ds(c * _HC, _HC)], zqb.at[par], zsems[par]
        ).start()

    def wait_in(par):
        pltpu.make_async_copy(
            zt_hbm.at[b, pl.ds(0, _HC)], zqb.at[par], zsems[par]
        ).wait()

    def start_out(c, par):
        pltpu.make_async_copy(
            zqb.at[par], qt_hbm.at[b, pl.ds(c * _HC, _HC)], qsems[par]
        ).start()
        pltpu.make_async_copy(
            ib.at[par], i_hbm.at[b, pl.ds(0, 8), pl.ds(c * _HC, _HC)],
            isems[par],
        ).start()

    def wait_out_q(par):
        pltpu.make_async_copy(
            zqb.at[par], qt_hbm.at[b, pl.ds(0, _HC)], qsems[par]
        ).wait()

    def wait_out_i(par):
        pltpu.make_async_copy(
            ib.at[par], i_hbm.at[b, pl.ds(0, 8), pl.ds(0, _HC)], isems[par]
        ).wait()

    start_in(0, 0)
    for c in range(_NCHUNK):
        par = c % 2
        if c + 1 < _NCHUNK:
            if c >= 1:
                wait_out_q(1 - par)
            start_in(c + 1, 1 - par)
        wait_in(par)
        if c >= 2:
            wait_out_i(par)
        _compute_chunk(zqb.at[par], scr, ib.at[par])
        start_out(c, par)
    wait_out_q(_NCHUNK % 2)
    wait_out_q((_NCHUNK - 1) % 2)
    wait_out_i(_NCHUNK % 2)
    wait_out_i((_NCHUNK - 1) % 2)


def kernel(z):
    B, D, H, W = z.shape
    zt = jnp.transpose(z, (0, 2, 3, 1))
    mesh = plsc.VectorSubcoreMesh(
        core_axis_name="c", subcore_axis_name="s",
        num_cores=_NC, num_subcores=_NS,
    )
    kfn = pl.kernel(
        _sc_body,
        out_type=(
            jax.ShapeDtypeStruct((B, H, W, D), jnp.float32),
            jax.ShapeDtypeStruct((B, D // 8, H, W), jnp.int32),
        ),
        mesh=mesh,
        scratch_types=[
            pltpu.VMEM((2, _HC, W, D), jnp.float32),
            pltpu.VMEM((_HC, W, D), jnp.int32),
            pltpu.VMEM((2, 8, _HC, W), jnp.int32),
            (pltpu.SemaphoreType.DMA, pltpu.SemaphoreType.DMA),
            (pltpu.SemaphoreType.DMA, pltpu.SemaphoreType.DMA),
            (pltpu.SemaphoreType.DMA, pltpu.SemaphoreType.DMA),
        ],
        compiler_params=pltpu.CompilerParams(
            use_tc_tiling_on_sc=True, needs_layout_passes=False,
        ),
    )
    qt, idx = kfn(zt)
    return jnp.transpose(qt, (0, 3, 1, 2)), idx
